# Initial kernel scaffold; baseline (speedup 1.0000x reference)
import functools

import jax
import jax.numpy as jnp
from jax import lax
from jax.experimental import pallas as pl
from jax.experimental.pallas import tpu as pltpu
from jax.experimental.pallas import tpu_sc as plsc

B, L, V, D = 4096, 26, 1000, 16
NC, NS, LANES = 2, 16, 16
NW = NC * NS
BPW = B // NW
NG = BPW // LANES


def _gather_sum(x3, table):
    mesh = plsc.VectorSubcoreMesh(core_axis_name="c", subcore_axis_name="s")

    @functools.partial(
        pl.kernel,
        out_type=jax.ShapeDtypeStruct((NW, D, BPW), jnp.float32),
        mesh=mesh,
        scratch_types=[
            pltpu.VMEM((V, D), jnp.float32),
            pltpu.VMEM((L, BPW), jnp.int32),
            pltpu.VMEM((D, BPW), jnp.float32),
        ],
    )
    def body(x_hbm, table_hbm, out_hbm, table_v, idx_v, s_v):
        wid = lax.axis_index("s") * NC + lax.axis_index("c")
        pltpu.sync_copy(table_hbm, table_v)
        pltpu.sync_copy(x_hbm.at[wid], idx_v)

        def group(g, carry):
            col = g * LANES
            accs = [jnp.zeros((LANES,), jnp.float32) for _ in range(D)]
            for l in range(L):
                xv = idx_v[l, pl.ds(col, LANES)]
                for d in range(D):
                    dv = jnp.full((LANES,), d, jnp.int32)
                    accs[d] += plsc.load_gather(table_v, [xv, dv])
            for d in range(D):
                s_v[d, pl.ds(col, LANES)] = accs[d]
            return carry

        lax.fori_loop(0, NG, group, 0)
        pltpu.sync_copy(s_v, out_hbm.at[wid])

    return body(x3, table)


def _decode(sT, W_dec):
    def mm(s_ref, w_ref, o_ref):
        s_blk = s_ref[0]
        o_ref[...] = lax.dot_general(
            s_blk, w_ref[...],
            dimension_numbers=(((0,), (1,)), ((), ())),
            preferred_element_type=jnp.float32,
            precision=lax.Precision.HIGHEST,
        )

    return pl.pallas_call(
        mm,
        grid=(NW,),
        in_specs=[
            pl.BlockSpec((1, D, BPW), lambda i: (i, 0, 0)),
            pl.BlockSpec((V, D), lambda i: (0, 0)),
        ],
        out_specs=pl.BlockSpec((BPW, V), lambda i: (i, 0)),
        out_shape=jax.ShapeDtypeStruct((B, V), jnp.float32),
    )(sT, W_dec)


def kernel(x, table, W_dec):
    x3 = x.astype(jnp.int32).reshape(NW, BPW, L).transpose(0, 2, 1)
    sT = _gather_sum(x3, table)
    return _decode(sT, W_dec)


# speedup vs baseline: 6.4732x; 6.4732x over previous
import functools

import jax
import jax.numpy as jnp
from jax import lax
from jax.experimental import pallas as pl
from jax.experimental.pallas import tpu as pltpu
from jax.experimental.pallas import tpu_sc as plsc

B, L, V, D = 4096, 26, 1000, 16
NC, NS, LANES = 2, 16, 16
NW = NC * NS
BPW = B // NW
NG = BPW // LANES


def _gather_sum(x3, table_flat):
    mesh = plsc.VectorSubcoreMesh(core_axis_name="c", subcore_axis_name="s")

    @functools.partial(
        pl.kernel,
        out_type=jax.ShapeDtypeStruct((NW, D, BPW), jnp.float32),
        mesh=mesh,
        scratch_types=[
            pltpu.VMEM((V * D,), jnp.float32),
            pltpu.VMEM((L, BPW), jnp.int32),
            pltpu.VMEM((D, BPW), jnp.float32),
        ],
        compiler_params=pltpu.CompilerParams(needs_layout_passes=False),
    )
    def body(x_hbm, table_hbm, out_hbm, table_v, idx_v, s_v):
        wid = lax.axis_index("s") * NC + lax.axis_index("c")
        pltpu.sync_copy(table_hbm, table_v)
        pltpu.sync_copy(x_hbm.at[wid], idx_v)

        def group(g, carry):
            col = g * LANES
            accs = [jnp.zeros((LANES,), jnp.float32) for _ in range(D)]
            for l in range(L):
                base = idx_v[l, pl.ds(col, LANES)] * D
                for d in range(D):
                    accs[d] += plsc.load_gather(table_v, [base + d])
            for d in range(D):
                s_v[d, pl.ds(col, LANES)] = accs[d]
            return carry

        lax.fori_loop(0, NG, group, 0)
        pltpu.sync_copy(s_v, out_hbm.at[wid])

    return body(x3, table_flat)


def _decode(sT, W_dec):
    def mm(s_ref, w_ref, o_ref):
        s_blk = s_ref[0]
        o_ref[...] = lax.dot_general(
            s_blk, w_ref[...],
            dimension_numbers=(((0,), (1,)), ((), ())),
            preferred_element_type=jnp.float32,
            precision=lax.Precision.HIGHEST,
        )

    return pl.pallas_call(
        mm,
        grid=(NW,),
        in_specs=[
            pl.BlockSpec((1, D, BPW), lambda i: (i, 0, 0)),
            pl.BlockSpec((V, D), lambda i: (0, 0)),
        ],
        out_specs=pl.BlockSpec((BPW, V), lambda i: (i, 0)),
        out_shape=jax.ShapeDtypeStruct((B, V), jnp.float32),
    )(sT, W_dec)


def kernel(x, table, W_dec):
    x3 = x.astype(jnp.int32).reshape(NW, BPW, L).transpose(0, 2, 1)
    sT = _gather_sum(x3, table.reshape(V * D))
    return _decode(sT, W_dec)
